# baseline (speedup 1.0000x reference)
import functools

import jax
import jax.numpy as jnp
from jax import lax
from jax.experimental import pallas as pl
from jax.experimental.pallas import tpu as pltpu, tpu_sc as plsc

N = 10000
E = 320000
D = 128
DA = 64
R = 4

NC = 2
NS = 16
NW = NC * NS
CHUNK = 128
NBUF = 2
EPW = -(-E // NW)
CPW = -(-(-(-EPW // CHUNK)) // NBUF) * NBUF
E_PAD = NW * CPW * CHUNK
ACC_ROWS = N + 8
ZCHUNK = 128
ZFULL = ACC_ROWS // ZCHUNK
ZTAIL = ACC_ROWS - ZFULL * ZCHUNK

_PREC = jax.lax.Precision.DEFAULT


def _dot(a, b):
    return lax.dot_general(a, b, (((1,), (1,)), ((), ())),
                           precision=_PREC, preferred_element_type=jnp.float32)


BN = 2000


def _init_body(f_ref, w_ref, b_ref, o_ref):
    o_ref[...] = _dot(f_ref[...], w_ref[...]) + b_ref[...]


def _mlp_body(h_ref, w1_ref, b1_ref, w2_ref, b2_ref, o_ref):
    t = jnp.maximum(_dot(h_ref[...], w1_ref[...]) + b1_ref[...], 0.0)
    o_ref[...] = _dot(t, w2_ref[...]) + b2_ref[...]


def _gru_body(p_ref, h_ref, wih_ref, whh_ref, bih_ref, bhh_ref, o_ref):
    x = p_ref[0] + p_ref[1]
    h = h_ref[...]
    gi = _dot(x, wih_ref[...]) + bih_ref[...]
    gh = _dot(h, whh_ref[...]) + bhh_ref[...]
    ir, iz, inn = gi[:, :D], gi[:, D:2 * D], gi[:, 2 * D:]
    hr, hz, hn = gh[:, :D], gh[:, D:2 * D], gh[:, 2 * D:]
    r = jax.nn.sigmoid(ir + hr)
    z = jax.nn.sigmoid(iz + hz)
    n = jnp.tanh(inn + r * hn)
    o_ref[...] = (1.0 - z) * n + z * h


def _row_spec(shape):
    return pl.BlockSpec((BN,) + shape[1:], lambda i: (0,) * 0 + (i,) + (0,) * (len(shape) - 1))


def _rep_spec(shape):
    return pl.BlockSpec(shape, lambda i: (0,) * len(shape))


def _tc_call(body, out_dim, in_shapes, row_in):
    in_specs = []
    for shp, is_row in zip(in_shapes, row_in):
        if is_row:
            if len(shp) == 3:
                in_specs.append(pl.BlockSpec((2, BN, shp[2]),
                                             lambda i: (0, i, 0)))
            else:
                in_specs.append(pl.BlockSpec((BN, shp[1]), lambda i: (i, 0)))
        else:
            in_specs.append(_rep_spec(shp))
    return pl.pallas_call(
        body,
        grid=(N // BN,),
        in_specs=in_specs,
        out_specs=pl.BlockSpec((BN, out_dim), lambda i: (i, 0)),
        out_shape=jax.ShapeDtypeStruct((N, out_dim), jnp.float32),
    )


_ZPT = -(-(ZFULL + 1) // NS)


@functools.cache
def _make_segsum_sc():
    mesh = plsc.VectorSubcoreMesh(core_axis_name="c", subcore_axis_name="s",
                                  num_cores=NC, num_subcores=NS)
    return functools.partial(
        pl.kernel,
        out_type=jax.ShapeDtypeStruct((NC, N, D), jnp.float32),
        mesh=mesh,
        scratch_types=[
            pltpu.VMEM((2, NBUF, CHUNK), jnp.int32),
            pltpu.VMEM((CPW, CHUNK), jnp.int32),
            pltpu.VMEM((NBUF, CHUNK, D), jnp.float32),
            pltpu.VMEM_SHARED((ACC_ROWS, D), jnp.float32),
            [pltpu.SemaphoreType.DMA] * 2,
            [pltpu.SemaphoreType.DMA] * NBUF,
            [pltpu.SemaphoreType.DMA] * NBUF,
        ],
    )(_segsum_body)


_G = CPW // NBUF


def _segsum_body(pre_hbm, gidx_hbm, sidx_hbm, z_hbm, out_hbm,
                 gw, sidx_v, rows_v, acc_s, isems, gsems, ssems):
    c = lax.axis_index("c")
    s = lax.axis_index("s")
    wid = c * NS + s

    pltpu.sync_copy(sidx_hbm.at[wid], sidx_v)
    pltpu.async_copy(gidx_hbm.at[wid, pl.ds(0, NBUF)], gw.at[0], isems[0])
    pltpu.async_copy(gidx_hbm.at[wid, pl.ds(NBUF, NBUF)], gw.at[1], isems[1])

    pltpu.make_async_copy(gidx_hbm.at[wid, pl.ds(0, NBUF)], gw.at[0],
                          isems[0]).wait()
    for b in range(NBUF):
        pltpu.async_copy(pre_hbm.at[gw.at[0, b]], rows_v.at[b], gsems[b])

    for k in range(_ZPT):
        j = s * _ZPT + k

        @pl.when(j < ZFULL)
        def _():
            pltpu.sync_copy(z_hbm, acc_s.at[pl.ds(j * ZCHUNK, ZCHUNK)])

        @pl.when(j == ZFULL)
        def _():
            pltpu.sync_copy(z_hbm.at[pl.ds(0, ZTAIL)],
                            acc_s.at[pl.ds(ZFULL * ZCHUNK, ZTAIL)])

    plsc.subcore_barrier()

    def _group(g, p, have_next, have_next2):
        for b in range(NBUF):
            j = g * NBUF + b
            pltpu.make_async_copy(pre_hbm.at[gw.at[p, b]], rows_v.at[b],
                                  gsems[b]).wait()
            pltpu.async_copy(rows_v.at[b], acc_s.at[sidx_v.at[j]], ssems[b],
                             add=True)
        @pl.when(have_next2)
        def _():
            pltpu.async_copy(gidx_hbm.at[wid, pl.ds((g + 2) * NBUF, NBUF)],
                             gw.at[p], isems[p])

        @pl.when(have_next)
        def _():
            pltpu.make_async_copy(
                gidx_hbm.at[wid, pl.ds((g + 1) * NBUF, NBUF)],
                gw.at[1 - p], isems[1 - p]).wait()
        for b in range(NBUF):
            j = g * NBUF + b

            @pl.when(have_next)
            def _():
                pltpu.make_async_copy(rows_v.at[b], acc_s.at[sidx_v.at[j]],
                                      ssems[b]).wait()
                pltpu.async_copy(pre_hbm.at[gw.at[1 - p, b]], rows_v.at[b],
                                 gsems[b])

    def step(i, carry):
        not_tail = i < _G // 2 - 1
        _group(2 * i, 0, jnp.bool_(True), not_tail)
        _group(2 * i + 1, 1, not_tail, not_tail)
        return carry

    lax.fori_loop(0, _G // 2, step, 0)

    for b in range(NBUF):
        j = CPW - NBUF + b
        pltpu.make_async_copy(rows_v.at[b], acc_s.at[sidx_v.at[j]],
                              ssems[b]).wait()

    plsc.subcore_barrier()

    n_full = N // ZCHUNK
    for k in range(_ZPT):
        j = s * _ZPT + k

        @pl.when(j < n_full)
        def _():
            pltpu.sync_copy(acc_s.at[pl.ds(j * ZCHUNK, ZCHUNK)],
                            out_hbm.at[c, pl.ds(j * ZCHUNK, ZCHUNK)])

        @pl.when(j == n_full)
        def _():
            pltpu.sync_copy(acc_s.at[pl.ds(n_full * ZCHUNK, N - n_full * ZCHUNK)],
                            out_hbm.at[c, pl.ds(n_full * ZCHUNK,
                                                N - n_full * ZCHUNK)])


def _pad_idx(idx, fill):
    pad = jnp.full((E_PAD - E,), fill, jnp.int32)
    return jnp.concatenate([idx, pad]).reshape(NW, CPW, CHUNK)


def kernel(features, edge_index, init_W, init_b,
           fmsg_W1, fmsg_b1, fmsg_W2, fmsg_b2,
           bmsg_W1, bmsg_b1, bmsg_W2, bmsg_b2,
           f_Wih, f_Whh, f_bih, f_bhh,
           b_Wih, b_Whh, b_bih, b_bhh):
    row = edge_index[0]
    col = edge_index[1]

    g_f = _pad_idx(col, 0)
    s_f = _pad_idx(row, N)
    g_b = _pad_idx(row, 0)
    s_b = _pad_idx(col, N)
    zblk = jnp.zeros((ZCHUNK, D), jnp.float32)

    init_b2 = init_b.reshape(1, D)
    fb1 = fmsg_b1.reshape(1, DA)
    fb2 = fmsg_b2.reshape(1, D)
    bb1 = bmsg_b1.reshape(1, DA)
    bb2 = bmsg_b2.reshape(1, D)
    fbih = f_bih.reshape(1, 3 * D)
    fbhh = f_bhh.reshape(1, 3 * D)
    bbih = b_bih.reshape(1, 3 * D)
    bbhh = b_bhh.reshape(1, 3 * D)

    init_call = _tc_call(_init_body, D,
                         [(N, 4), (D, 4), (1, D)], [True, False, False])
    mlp_call = _tc_call(_mlp_body, D,
                        [(N, D), (DA, D), (1, DA), (D, DA), (1, D)],
                        [True, False, False, False, False])
    gru_call = _tc_call(_gru_body, D,
                        [(NC, N, D), (N, D), (3 * D, D), (3 * D, D),
                         (1, 3 * D), (1, 3 * D)],
                        [True, True, False, False, False, False])

    h = init_call(features, init_W, init_b2)
    for _ in range(R):
        f_pre = mlp_call(h, fmsg_W1, fb1, fmsg_W2, fb2)
        f_part = _make_segsum_sc()(f_pre, g_f, s_f, zblk)
        h = gru_call(f_part, h, f_Wih, f_Whh, fbih, fbhh)
        b_pre = mlp_call(h, bmsg_W1, bb1, bmsg_W2, bb2)
        b_part = _make_segsum_sc()(b_pre, g_b, s_b, zblk)
        h = gru_call(b_part, h, b_Wih, b_Whh, bbih, bbhh)
    return h


# speedup vs baseline: 1.3741x; 1.3741x over previous
import functools

import jax
import jax.numpy as jnp
from jax import lax
from jax.experimental import pallas as pl
from jax.experimental.pallas import tpu as pltpu, tpu_sc as plsc

N = 10000
E = 320000
D = 128
DA = 64
R = 4

NC = 2
NS = 16
NW = NC * NS
CHUNK = 128
EPW = -(-E // NW)
CPW = -(-EPW // CHUNK)
E_PAD = NW * CPW * CHUNK
ACC_ROWS = CPW * CHUNK
ZCHUNK = 128
ZFULL = ACC_ROWS // ZCHUNK

_PREC = jax.lax.Precision.DEFAULT


def _dot(a, b):
    return lax.dot_general(a, b, (((1,), (1,)), ((), ())),
                           precision=_PREC, preferred_element_type=jnp.float32)


BN = 2000


def _init_body(f_ref, w_ref, b_ref, o_ref):
    o_ref[...] = _dot(f_ref[...], w_ref[...]) + b_ref[...]


def _mlp_body(h_ref, w1_ref, b1_ref, w2_ref, b2_ref, o_ref):
    t = jnp.maximum(_dot(h_ref[...], w1_ref[...]) + b1_ref[...], 0.0)
    o_ref[...] = _dot(t, w2_ref[...]) + b2_ref[...]


def _gru_body(p_ref, h_ref, wih_ref, whh_ref, bih_ref, bhh_ref, o_ref):
    x = p_ref[0] + p_ref[1]
    h = h_ref[...]
    gi = _dot(x, wih_ref[...]) + bih_ref[...]
    gh = _dot(h, whh_ref[...]) + bhh_ref[...]
    ir, iz, inn = gi[:, :D], gi[:, D:2 * D], gi[:, 2 * D:]
    hr, hz, hn = gh[:, :D], gh[:, D:2 * D], gh[:, 2 * D:]
    r = jax.nn.sigmoid(ir + hr)
    z = jax.nn.sigmoid(iz + hz)
    n = jnp.tanh(inn + r * hn)
    o_ref[...] = (1.0 - z) * n + z * h


def _row_spec(shape):
    return pl.BlockSpec((BN,) + shape[1:], lambda i: (0,) * 0 + (i,) + (0,) * (len(shape) - 1))


def _rep_spec(shape):
    return pl.BlockSpec(shape, lambda i: (0,) * len(shape))


def _tc_call(body, out_dim, in_shapes, row_in):
    in_specs = []
    for shp, is_row in zip(in_shapes, row_in):
        if is_row:
            if len(shp) == 3:
                in_specs.append(pl.BlockSpec((2, BN, shp[2]),
                                             lambda i: (0, i, 0)))
            else:
                in_specs.append(pl.BlockSpec((BN, shp[1]), lambda i: (i, 0)))
        else:
            in_specs.append(_rep_spec(shp))
    return pl.pallas_call(
        body,
        grid=(N // BN,),
        in_specs=in_specs,
        out_specs=pl.BlockSpec((BN, out_dim), lambda i: (i, 0)),
        out_shape=jax.ShapeDtypeStruct((N, out_dim), jnp.float32),
    )


_ZPT = -(-(ZFULL + 1) // NS)


@functools.cache
def _make_segsum_sc():
    mesh = plsc.VectorSubcoreMesh(core_axis_name="c", subcore_axis_name="s",
                                  num_cores=NC, num_subcores=NS)
    return functools.partial(
        pl.kernel,
        out_type=jax.ShapeDtypeStruct((NC, N, D), jnp.float32),
        mesh=mesh,
        scratch_types=[
            pltpu.VMEM((CPW, CHUNK), jnp.int32),
            pltpu.VMEM((CPW, CHUNK), jnp.int32),
            pltpu.VMEM((CHUNK, D), jnp.float32),
            pltpu.VMEM_SHARED((ACC_ROWS, D), jnp.float32),
            pltpu.SemaphoreType.DMA,
        ],
    )(_segsum_body)


def _segsum_body(pre_hbm, gidx_hbm, sidx_hbm, z_hbm, out_hbm,
                 gidx_v, sidx_v, rows_v, acc_s, sem):
    c = lax.axis_index("c")
    s = lax.axis_index("s")
    wid = c * NS + s

    for k in range(_ZPT):
        j = s * _ZPT + k

        @pl.when(j < ZFULL)
        def _():
            pltpu.sync_copy(z_hbm, acc_s.at[pl.ds(j * ZCHUNK, ZCHUNK)])

    plsc.subcore_barrier()

    pltpu.sync_copy(gidx_hbm.at[wid], gidx_v)
    pltpu.sync_copy(sidx_hbm.at[wid], sidx_v)

    def body(j, carry):
        pltpu.async_copy(pre_hbm.at[gidx_v.at[j]], rows_v, sem).wait()
        pltpu.sync_copy(rows_v, acc_s.at[sidx_v.at[j]], add=True)
        return carry

    lax.fori_loop(0, CPW, body, 0)

    plsc.subcore_barrier()

    n_full = N // ZCHUNK
    for k in range(_ZPT):
        j = s * _ZPT + k

        @pl.when(j < n_full)
        def _():
            pltpu.sync_copy(acc_s.at[pl.ds(j * ZCHUNK, ZCHUNK)],
                            out_hbm.at[c, pl.ds(j * ZCHUNK, ZCHUNK)])

        @pl.when(j == n_full)
        def _():
            pltpu.sync_copy(acc_s.at[pl.ds(n_full * ZCHUNK, N - n_full * ZCHUNK)],
                            out_hbm.at[c, pl.ds(n_full * ZCHUNK,
                                                N - n_full * ZCHUNK)])


def _pad_idx(idx, fill):
    pad = jnp.full((E_PAD - E,), fill, jnp.int32)
    return jnp.concatenate([idx, pad]).reshape(NW, CPW, CHUNK)


def kernel(features, edge_index, init_W, init_b,
           fmsg_W1, fmsg_b1, fmsg_W2, fmsg_b2,
           bmsg_W1, bmsg_b1, bmsg_W2, bmsg_b2,
           f_Wih, f_Whh, f_bih, f_bhh,
           b_Wih, b_Whh, b_bih, b_bhh):
    row = edge_index[0]
    col = edge_index[1]

    g_f = _pad_idx(col, 0)
    s_f = _pad_idx(row, N)
    g_b = _pad_idx(row, 0)
    s_b = _pad_idx(col, N)
    zblk = jnp.zeros((ZCHUNK, D), jnp.float32)

    init_b2 = init_b.reshape(1, D)
    fb1 = fmsg_b1.reshape(1, DA)
    fb2 = fmsg_b2.reshape(1, D)
    bb1 = bmsg_b1.reshape(1, DA)
    bb2 = bmsg_b2.reshape(1, D)
    fbih = f_bih.reshape(1, 3 * D)
    fbhh = f_bhh.reshape(1, 3 * D)
    bbih = b_bih.reshape(1, 3 * D)
    bbhh = b_bhh.reshape(1, 3 * D)

    init_call = _tc_call(_init_body, D,
                         [(N, 4), (D, 4), (1, D)], [True, False, False])
    mlp_call = _tc_call(_mlp_body, D,
                        [(N, D), (DA, D), (1, DA), (D, DA), (1, D)],
                        [True, False, False, False, False])
    gru_call = _tc_call(_gru_body, D,
                        [(NC, N, D), (N, D), (3 * D, D), (3 * D, D),
                         (1, 3 * D), (1, 3 * D)],
                        [True, True, False, False, False, False])

    h = init_call(features, init_W, init_b2)
    for _ in range(R):
        f_pre = mlp_call(h, fmsg_W1, fb1, fmsg_W2, fb2)
        f_part = _make_segsum_sc()(f_pre, g_f, s_f, zblk)
        h = gru_call(f_part, h, f_Wih, f_Whh, fbih, fbhh)
        b_pre = mlp_call(h, bmsg_W1, bb1, bmsg_W2, bb2)
        b_part = _make_segsum_sc()(b_pre, g_b, s_b, zblk)
        h = gru_call(b_part, h, b_Wih, b_Whh, bbih, bbhh)
    return h
